# baseline (speedup 1.0000x reference)
import functools

import jax
import jax.numpy as jnp
from jax import lax
from jax.experimental import pallas as pl
from jax.experimental.pallas import tpu as pltpu
from jax.experimental.pallas import tpu_sc as plsc

N = 10000
E = 320000
D = 128
H = 64
EPS = 1e-5

NC = 2
NS = 16
NW = NC * NS
CH = 128
EPW = E // NW
NFULL = EPW // CH
TAIL = EPW - NFULL * CH

_MESH = dict(core_axis_name="c", subcore_axis_name="s", num_cores=NC,
             num_subcores=NS)

ROWS_PER_SUB = N // NS


def _sc_gather(T, ei1d):
    @functools.partial(
        pl.kernel,
        out_type=jax.ShapeDtypeStruct((E, H), jnp.float32),
        mesh=plsc.VectorSubcoreMesh(**_MESH),
        scratch_types=[
            pltpu.VMEM((EPW,), jnp.int32),
            pltpu.VMEM((EPW,), jnp.int32),
            pltpu.VMEM((CH, H), jnp.float32),
            pltpu.VMEM((CH, H), jnp.float32),
            pltpu.VMEM((TAIL, H), jnp.float32),
            pltpu.SemaphoreType.DMA,
            pltpu.SemaphoreType.DMA,
        ],
        compiler_params=pltpu.CompilerParams(use_tc_tiling_on_sc=False),
    )
    def k(t_hbm, ei_hbm, pre_hbm, idxd, idxs, buf0, buf1, buf_t,
          sem0, sem1):
        c = lax.axis_index("c")
        s = lax.axis_index("s")
        wid = s * NC + c
        base0 = wid * EPW

        pltpu.sync_copy(ei_hbm.at[pl.ds(E + base0, EPW)], idxd)
        pltpu.sync_copy(ei_hbm.at[pl.ds(base0, EPW)], idxs)

        def shift(i, carry):
            sl = pl.ds(i * 16, 16)
            idxd[sl] = idxd[sl] * 2
            idxs[sl] = idxs[sl] * 2 + 1
            return carry

        lax.fori_loop(0, EPW // 16, shift, 0)

        def gsta(ci, b, sem, add):
            iref = idxs if add else idxd
            pltpu.async_copy(t_hbm.at[iref.at[pl.ds(ci * CH, CH)]], b, sem,
                             add=add)

        def gwait(ci, b, sem, add):
            iref = idxs if add else idxd
            pltpu.make_async_copy(t_hbm.at[iref.at[pl.ds(ci * CH, CH)]], b,
                                  sem).wait()

        def ssta(ci, b, sem):
            pltpu.async_copy(b, pre_hbm.at[pl.ds(base0 + ci * CH, CH)], sem)

        def swait(ci, b, sem):
            pltpu.make_async_copy(b, pre_hbm.at[pl.ds(base0 + ci * CH, CH)],
                                  sem).wait()

        NP = NFULL // 2
        gsta(0, buf0, sem0, False)

        def body(i, carry):
            c0 = 2 * i
            c1 = c0 + 1
            gwait(c0, buf0, sem0, False)
            gsta(c0, buf0, sem0, True)

            @pl.when(i > 0)
            def _():
                swait(c1 - 2, buf1, sem1)

            gsta(c1, buf1, sem1, False)
            gwait(c0, buf0, sem0, True)
            ssta(c0, buf0, sem0)
            gwait(c1, buf1, sem1, False)
            gsta(c1, buf1, sem1, True)
            swait(c0, buf0, sem0)

            @pl.when(i < NP - 1)
            def _():
                gsta(c0 + 2, buf0, sem0, False)

            gwait(c1, buf1, sem1, True)
            ssta(c1, buf1, sem1)
            return carry

        lax.fori_loop(0, NP, body, 0)
        swait(NFULL - 1, buf1, sem1)

        tb = NFULL * CH
        pltpu.sync_copy(t_hbm.at[idxd.at[pl.ds(tb, TAIL)]], buf_t)
        pltpu.sync_copy(t_hbm.at[idxs.at[pl.ds(tb, TAIL)]], buf_t, add=True)
        pltpu.sync_copy(buf_t, pre_hbm.at[pl.ds(base0 + tb, TAIL)])

    return k(T, ei1d)


DW = 16

ES = E // 2
EPWS = ES // NW
NFS = EPWS // CH
TAILS = EPWS - NFS * CH


def _sc_scatter(h2, ei1d, with_deg, part):
    out_type = [jax.ShapeDtypeStruct((NC, N, H), jnp.float32)]
    scratch = [
        pltpu.VMEM_SHARED((N, H), jnp.float32),
        pltpu.VMEM((ROWS_PER_SUB, H), jnp.float32),
        pltpu.VMEM((CH,), jnp.int32),
        pltpu.VMEM((CH,), jnp.int32),
        pltpu.VMEM((CH, H), jnp.float32),
        pltpu.VMEM((CH, H), jnp.float32),
        pltpu.VMEM((TAILS,), jnp.int32),
        pltpu.VMEM((TAILS, H), jnp.float32),
        pltpu.SemaphoreType.DMA,
        pltpu.SemaphoreType.DMA,
        pltpu.SemaphoreType.DMA,
        pltpu.SemaphoreType.DMA,
    ]
    if with_deg:
        out_type.append(jax.ShapeDtypeStruct((NC, N, DW), jnp.float32))
        scratch += [
            pltpu.VMEM_SHARED((N, DW), jnp.float32),
            pltpu.VMEM((ROWS_PER_SUB, DW), jnp.float32),
            pltpu.VMEM((CH, DW), jnp.float32),
            pltpu.VMEM((TAILS, DW), jnp.float32),
        ]

    @functools.partial(
        pl.kernel,
        out_type=tuple(out_type),
        mesh=plsc.VectorSubcoreMesh(**_MESH),
        scratch_types=scratch,
        compiler_params=pltpu.CompilerParams(use_tc_tiling_on_sc=False),
    )
    def k(h2_hbm, ei_hbm, out_hbm, *rest):
        if with_deg:
            (outd_hbm, acc, stage, idx0, idx1, buf0, buf1, idx_t, buf_t,
             semL0, semL1, semW0, semW1, dacc, dstage, ones, ones_t) = rest
        else:
            (acc, stage, idx0, idx1, buf0, buf1, idx_t, buf_t,
             semL0, semL1, semW0, semW1) = rest
        c = lax.axis_index("c")
        s = lax.axis_index("s")
        wid = s * NC + c
        base0 = wid * EPWS
        eibase = E + part * ES
        rsl = pl.ds(s * ROWS_PER_SUB, ROWS_PER_SUB)

        def zrow(i, carry):
            for j in range(H // 16):
                stage[i, pl.ds(j * 16, 16)] = jnp.zeros((16,), jnp.float32)
            return carry

        lax.fori_loop(0, ROWS_PER_SUB, zrow, 0)
        pltpu.sync_copy(stage, acc.at[rsl])
        if with_deg:
            def zdrow(i, carry):
                dstage[i, pl.ds(0, 16)] = jnp.zeros((16,), jnp.float32)
                return carry

            lax.fori_loop(0, ROWS_PER_SUB, zdrow, 0)
            pltpu.sync_copy(dstage, dacc.at[rsl])

            def orow(i, carry):
                ones[i, pl.ds(0, 16)] = jnp.ones((16,), jnp.float32)
                return carry

            lax.fori_loop(0, CH, orow, 0)
            for i in range(TAILS):
                ones_t[i, pl.ds(0, 16)] = jnp.ones((16,), jnp.float32)
        plsc.subcore_barrier()

        def lsta(ci, i_r, b, sem):
            base = base0 + ci * CH
            pltpu.async_copy(ei_hbm.at[pl.ds(eibase + base, CH)], i_r, sem)
            pltpu.async_copy(h2_hbm.at[pl.ds(base, CH)], b, sem)

        def lwait(ci, i_r, b, sem):
            base = base0 + ci * CH
            pltpu.make_async_copy(ei_hbm.at[pl.ds(eibase + base, CH)], i_r,
                                  sem).wait()
            pltpu.make_async_copy(h2_hbm.at[pl.ds(base, CH)], b, sem).wait()

        def wsta(i_r, b, sem):
            pltpu.async_copy(b, acc.at[i_r], sem, add=True)

        def wwait(i_r, b, sem):
            pltpu.make_async_copy(b, acc.at[i_r], sem).wait()

        NP = NFS // 2
        lsta(0, idx0, buf0, semL0)

        def body(i, carry):
            c0 = 2 * i
            c1 = c0 + 1
            lwait(c0, idx0, buf0, semL0)

            @pl.when(i > 0)
            def _():
                wwait(idx1, buf1, semW1)

            lsta(c1, idx1, buf1, semL1)
            wsta(idx0, buf0, semW0)
            if with_deg:
                pltpu.sync_copy(ones, dacc.at[idx0], add=True)
            lwait(c1, idx1, buf1, semL1)
            wwait(idx0, buf0, semW0)
            lsta(c0 + 2, idx0, buf0, semL0)
            wsta(idx1, buf1, semW1)
            if with_deg:
                pltpu.sync_copy(ones, dacc.at[idx1], add=True)
            return carry

        lax.fori_loop(0, NP, body, 0)
        lwait(NFS - 1, idx0, buf0, semL0)
        wwait(idx1, buf1, semW1)
        wsta(idx0, buf0, semW0)
        if with_deg:
            pltpu.sync_copy(ones, dacc.at[idx0], add=True)
        wwait(idx0, buf0, semW0)

        tb = base0 + NFS * CH
        pltpu.sync_copy(ei_hbm.at[pl.ds(eibase + tb, TAILS)], idx_t)
        pltpu.sync_copy(h2_hbm.at[pl.ds(tb, TAILS)], buf_t)
        pltpu.sync_copy(buf_t, acc.at[idx_t], add=True)
        if with_deg:
            pltpu.sync_copy(ones_t, dacc.at[idx_t], add=True)

        plsc.subcore_barrier()
        pltpu.sync_copy(acc.at[rsl], stage)
        pltpu.sync_copy(stage, out_hbm.at[c, rsl])
        if with_deg:
            pltpu.sync_copy(dacc.at[rsl], dstage)
            pltpu.sync_copy(dstage, outd_hbm.at[c, rsl])

    res = k(h2, ei1d)
    return res if with_deg else res[0]


def _prep1(x, w1):
    BR, NB = 2000, N // 2000

    def body(x_ref, w_ref, o_ref):
        w = w_ref[...]
        wb = w[D:]
        wcat = jnp.concatenate([w[:D] - wb, wb], axis=1)
        o_ref[...] = jnp.dot(x_ref[...], wcat,
                             preferred_element_type=jnp.float32)

    return pl.pallas_call(
        body,
        grid=(NB,),
        in_specs=[
            pl.BlockSpec((BR, D), lambda j: (j, 0)),
            pl.BlockSpec((2 * D, H), lambda j: (0, 0)),
        ],
        out_specs=pl.BlockSpec((BR, 2 * H), lambda j: (j, 0)),
        out_shape=jax.ShapeDtypeStruct((N, 2 * H), jnp.float32),
    )(x, w1)


def _prep2(S, w1, s2d, crow, degp):
    BR, NB = 2000, N // 2000

    def body(s_ref, w_ref, sc_ref, c_ref, d_ref, o_ref):
        xin = s_ref[0] + s_ref[1]
        dcol = d_ref[0, :, 0:1] + d_ref[1, :, 0:1]
        w = w_ref[...]
        wb = w[H:]
        wcat = jnp.concatenate([w[:H] - wb, wb], axis=1)
        vrow = jnp.dot(c_ref[...], wcat, preferred_element_type=jnp.float32)
        o_ref[...] = (jnp.dot(xin, sc_ref[...] * wcat,
                              preferred_element_type=jnp.float32)
                      + dcol * vrow)

    return pl.pallas_call(
        body,
        grid=(NB,),
        in_specs=[
            pl.BlockSpec((2, BR, H), lambda j: (0, j, 0)),
            pl.BlockSpec((2 * H, H), lambda j: (0, 0)),
            pl.BlockSpec((H, 1), lambda j: (0, 0)),
            pl.BlockSpec((1, H), lambda j: (0, 0)),
            pl.BlockSpec((2, BR, DW), lambda j: (0, j, 0)),
        ],
        out_specs=pl.BlockSpec((BR, 2 * H), lambda j: (j, 0)),
        out_shape=jax.ShapeDtypeStruct((N, 2 * H), jnp.float32),
    )(S, w1, s2d, crow, degp)


def _edge_stats(pre128, b1row):
    BR = 8000
    E2 = E // 2
    NB = E2 // BR

    def body(p_ref, b_ref, o_ref):
        b128 = jnp.concatenate([b_ref[...], b_ref[...]], axis=1)
        h = jnp.maximum(p_ref[...] + b128, 0.0)

        @pl.when(pl.program_id(0) == 0)
        def _():
            o_ref[...] = jnp.zeros_like(o_ref)

        o_ref[0:1, :] += jnp.sum(h, axis=0, keepdims=True)
        o_ref[1:2, :] += jnp.sum(h * h, axis=0, keepdims=True)

    return pl.pallas_call(
        body,
        grid=(NB,),
        in_specs=[
            pl.BlockSpec((BR, 2 * H), lambda i: (i, 0)),
            pl.BlockSpec((1, H), lambda i: (0, 0)),
        ],
        out_specs=pl.BlockSpec((8, 2 * H), lambda i: (0, 0)),
        out_shape=jax.ShapeDtypeStruct((8, 2 * H), jnp.float32),
    )(pre128, b1row)


def _edge_mm(pre128, b1row, w2, s1d, c1row, b2row, part):
    BR = 8000
    E2 = ES // 2
    NB = E2 // BR

    def body(p_ref, b1_ref, w_ref, s1_ref, c1_ref, b2_ref, h2_ref, st_ref):
        w2p = s1_ref[...] * w_ref[...]
        b2p = (jnp.dot(c1_ref[...], w_ref[...],
                       preferred_element_type=jnp.float32) + b2_ref[...])
        b128 = jnp.concatenate([b1_ref[...], b1_ref[...]], axis=1)
        h1 = jnp.maximum(p_ref[...] + b128, 0.0)
        zz = jnp.zeros((H, H), jnp.float32)
        w2bd = jnp.concatenate(
            [jnp.concatenate([w2p, zz], axis=1),
             jnp.concatenate([zz, w2p], axis=1)], axis=0)
        h2 = jnp.maximum(
            jnp.dot(h1, w2bd, preferred_element_type=jnp.float32)
            + jnp.concatenate([b2p, b2p], axis=1), 0.0)
        h2_ref[...] = h2

        @pl.when(pl.program_id(0) == 0)
        def _():
            st_ref[...] = jnp.zeros_like(st_ref)

        st_ref[0:1, :] += jnp.sum(h2, axis=0, keepdims=True)
        st_ref[1:2, :] += jnp.sum(h2 * h2, axis=0, keepdims=True)

    return pl.pallas_call(
        body,
        grid=(NB,),
        in_specs=[
            pl.BlockSpec((BR, 2 * H), lambda i: (i + part * NB, 0)),
            pl.BlockSpec((1, H), lambda i: (0, 0)),
            pl.BlockSpec((H, H), lambda i: (0, 0)),
            pl.BlockSpec((H, 1), lambda i: (0, 0)),
            pl.BlockSpec((1, H), lambda i: (0, 0)),
            pl.BlockSpec((1, H), lambda i: (0, 0)),
        ],
        out_specs=[
            pl.BlockSpec((BR, 2 * H), lambda i: (i, 0)),
            pl.BlockSpec((8, 2 * H), lambda i: (0, 0)),
        ],
        out_shape=[
            jax.ShapeDtypeStruct((E2, 2 * H), jnp.float32),
            jax.ShapeDtypeStruct((8, 2 * H), jnp.float32),
        ],
    )(pre128, b1row, w2, s1d, c1row, b2row)


def _cat_mm(S1, S2, degp, w, brow, sa, sb, ca, cb):
    BR, NB = 2000, N // 2000
    MH = w.shape[1]

    def body(s1_ref, s2_ref, d_ref, w_ref, b_ref, sa_ref, sb_ref, ca_ref,
             cb_ref, r_ref, st_ref):
        x1 = s1_ref[0] + s1_ref[1]
        x2 = s2_ref[0] + s2_ref[1]
        dcol = d_ref[0, :, 0:1] + d_ref[1, :, 0:1]
        wt = w_ref[...][:H]
        wb = w_ref[...][H:]
        vrow = (jnp.dot(ca_ref[...], wt, preferred_element_type=jnp.float32)
                + jnp.dot(cb_ref[...], wb,
                          preferred_element_type=jnp.float32))
        z = (jnp.dot(x1, sa_ref[...] * wt,
                     preferred_element_type=jnp.float32)
             + jnp.dot(x2, sb_ref[...] * wb,
                       preferred_element_type=jnp.float32)
             + dcol * vrow + b_ref[...])
        r = jnp.maximum(z, 0.0)
        r_ref[...] = r

        @pl.when(pl.program_id(0) == 0)
        def _():
            st_ref[...] = jnp.zeros_like(st_ref)

        st_ref[0:1, :] += jnp.sum(r, axis=0, keepdims=True)
        st_ref[1:2, :] += jnp.sum(r * r, axis=0, keepdims=True)

    return pl.pallas_call(
        body,
        grid=(NB,),
        in_specs=[
            pl.BlockSpec((2, BR, H), lambda j: (0, j, 0)),
            pl.BlockSpec((2, BR, H), lambda j: (0, j, 0)),
            pl.BlockSpec((2, BR, DW), lambda j: (0, j, 0)),
            pl.BlockSpec((2 * H, MH), lambda j: (0, 0)),
            pl.BlockSpec((1, MH), lambda j: (0, 0)),
            pl.BlockSpec((H, 1), lambda j: (0, 0)),
            pl.BlockSpec((H, 1), lambda j: (0, 0)),
            pl.BlockSpec((1, H), lambda j: (0, 0)),
            pl.BlockSpec((1, H), lambda j: (0, 0)),
        ],
        out_specs=[
            pl.BlockSpec((BR, MH), lambda j: (j, 0)),
            pl.BlockSpec((8, MH), lambda j: (0, 0)),
        ],
        out_shape=[
            jax.ShapeDtypeStruct((N, MH), jnp.float32),
            jax.ShapeDtypeStruct((8, MH), jnp.float32),
        ],
    )(S1, S2, degp, w, brow, sa, sb, ca, cb)


def _node_mm(xin, w, brow, sd, crow):
    BR, NB = 2000, N // 2000
    K, M = w.shape

    def body(x_ref, w_ref, b_ref, s_ref, c_ref, r_ref, st_ref):
        w = w_ref[...]
        beff = (jnp.dot(c_ref[...], w, preferred_element_type=jnp.float32)
                + b_ref[...])
        r = jnp.maximum(
            jnp.dot(x_ref[...], s_ref[...] * w,
                    preferred_element_type=jnp.float32) + beff, 0.0)
        r_ref[...] = r

        @pl.when(pl.program_id(0) == 0)
        def _():
            st_ref[...] = jnp.zeros_like(st_ref)

        st_ref[0:1, :] += jnp.sum(r, axis=0, keepdims=True)
        st_ref[1:2, :] += jnp.sum(r * r, axis=0, keepdims=True)

    return pl.pallas_call(
        body,
        grid=(NB,),
        in_specs=[
            pl.BlockSpec((BR, K), lambda j: (j, 0)),
            pl.BlockSpec((K, M), lambda j: (0, 0)),
            pl.BlockSpec((1, M), lambda j: (0, 0)),
            pl.BlockSpec((K, 1), lambda j: (0, 0)),
            pl.BlockSpec((1, K), lambda j: (0, 0)),
        ],
        out_specs=[
            pl.BlockSpec((BR, M), lambda j: (j, 0)),
            pl.BlockSpec((8, M), lambda j: (0, 0)),
        ],
        out_shape=[
            jax.ShapeDtypeStruct((N, M), jnp.float32),
            jax.ShapeDtypeStruct((8, M), jnp.float32),
        ],
    )(xin, w, brow, sd, crow)


def _final_mm(xin, w, brow, sd, crow):
    BR, NB = 2000, N // 2000
    K, M = w.shape

    def body(x_ref, w_ref, b_ref, s_ref, c_ref, o_ref):
        w = w_ref[...]
        beff = (jnp.dot(c_ref[...], w, preferred_element_type=jnp.float32)
                + b_ref[...])
        z = jnp.dot(x_ref[...], s_ref[...] * w,
                    preferred_element_type=jnp.float32) + beff
        m = jnp.max(z, axis=1, keepdims=True)
        lse = jnp.log(jnp.sum(jnp.exp(z - m), axis=1, keepdims=True)) + m
        o_ref[...] = z - lse

    return pl.pallas_call(
        body,
        grid=(NB,),
        in_specs=[
            pl.BlockSpec((BR, K), lambda j: (j, 0)),
            pl.BlockSpec((K, M), lambda j: (0, 0)),
            pl.BlockSpec((1, M), lambda j: (0, 0)),
            pl.BlockSpec((K, 1), lambda j: (0, 0)),
            pl.BlockSpec((1, K), lambda j: (0, 0)),
        ],
        out_specs=pl.BlockSpec((BR, M), lambda j: (j, 0)),
        out_shape=jax.ShapeDtypeStruct((N, M), jnp.float32),
    )(xin, w, brow, sd, crow)


def _bn_fold(st, g, be):
    mu = st[0] / E
    var = st[1] / E - mu * mu
    s = g / jnp.sqrt(var + EPS)
    return mu, s, be - s * mu


def _bn_fold_n(st, g, be):
    mu = st[0] / N
    var = st[1] / N - mu * mu
    s = g / jnp.sqrt(var + EPS)
    return mu, s, be - s * mu


def _edge_layer(xin_T, ei1d, b1, g1, be1, W2, b2, g2, be2, with_deg):
    pre = _sc_gather(xin_T, ei1d)
    pre128 = pre.reshape(E // 2, 2 * H)
    b1r = b1.reshape(1, H)
    st1p = _edge_stats(pre128, b1r)
    st1 = st1p[:, :H] + st1p[:, H:]
    _, s1, c1 = _bn_fold(st1, g1, be1)
    s1d = s1.reshape(H, 1)
    c1r = c1.reshape(1, H)
    b2r = b2.reshape(1, H)
    h2a, st2pa = _edge_mm(pre128, b1r, W2, s1d, c1r, b2r, 0)
    h2b, st2pb = _edge_mm(pre128, b1r, W2, s1d, c1r, b2r, 1)
    st2p = st2pa + st2pb
    st2 = st2p[:, :H] + st2p[:, H:]
    mu2 = st2[0] / E
    var2 = st2[1] / E - mu2 * mu2
    s2 = g2 / jnp.sqrt(var2 + EPS)
    c2 = be2 - s2 * mu2
    outa = _sc_scatter(h2a.reshape(ES, H), ei1d, with_deg, 0)
    outb = _sc_scatter(h2b.reshape(ES, H), ei1d, with_deg, 1)
    if with_deg:
        S = outa[0] + outb[0]
        degp = outa[1] + outb[1]
        return S, s2, c2, degp
    return outa + outb, s2, c2


def kernel(x, edge_index, c1_W1, c1_b1, c1_g1, c1_be1, c1_W2, c1_b2, c1_g2,
           c1_be2, c2_W1, c2_b1, c2_g1, c2_be1, c2_W2, c2_b2, c2_g2, c2_be2,
           l1_W, l1_b, l1_g, l1_be, m1_W, m1_b, m1_g, m1_be, m2_W, m2_b,
           m2_g, m2_be, f_W, f_b):
    ei1d = edge_index.reshape(2 * E)

    T1 = _prep1(x, c1_W1).reshape(2 * N, H)
    S1, s2a, c2a, degp = _edge_layer(T1, ei1d, c1_b1, c1_g1, c1_be1,
                                     c1_W2, c1_b2, c1_g2, c1_be2, True)

    T2 = _prep2(S1, c2_W1, s2a.reshape(H, 1), c2a.reshape(1, H),
                degp).reshape(2 * N, H)
    S2, s2b, c2b = _edge_layer(T2, ei1d, c2_b1, c2_g1, c2_be1,
                               c2_W2, c2_b2, c2_g2, c2_be2, False)

    r1, stA = _cat_mm(S1, S2, degp, l1_W, l1_b.reshape(1, -1),
                      s2a.reshape(H, 1), s2b.reshape(H, 1),
                      c2a.reshape(1, H), c2b.reshape(1, H))
    _, sA, cA = _bn_fold_n(stA, l1_g, l1_be)

    r2, stB = _node_mm(r1, m1_W, m1_b.reshape(1, -1),
                       sA.reshape(-1, 1), cA.reshape(1, -1))
    _, sB, cB = _bn_fold_n(stB, m1_g, m1_be)

    r3, stC = _node_mm(r2, m2_W, m2_b.reshape(1, -1),
                       sB.reshape(-1, 1), cB.reshape(1, -1))
    _, sC, cC = _bn_fold_n(stC, m2_g, m2_be)

    return _final_mm(r3, f_W, f_b.reshape(1, -1),
                     sC.reshape(-1, 1), cC.reshape(1, -1))


# speedup vs baseline: 1.0647x; 1.0647x over previous
import functools

import jax
import jax.numpy as jnp
from jax import lax
from jax.experimental import pallas as pl
from jax.experimental.pallas import tpu as pltpu
from jax.experimental.pallas import tpu_sc as plsc

N = 10000
E = 320000
D = 128
H = 64
EPS = 1e-5

NC = 2
NS = 16
NW = NC * NS
CH = 128
EPW = E // NW
NFULL = EPW // CH
TAIL = EPW - NFULL * CH

_MESH = dict(core_axis_name="c", subcore_axis_name="s", num_cores=NC,
             num_subcores=NS)

ROWS_PER_SUB = N // NS


def _sc_gather(T, ei1d):
    @functools.partial(
        pl.kernel,
        out_type=jax.ShapeDtypeStruct((E, H), jnp.float32),
        mesh=plsc.VectorSubcoreMesh(**_MESH),
        scratch_types=[
            pltpu.VMEM((EPW,), jnp.int32),
            pltpu.VMEM((EPW,), jnp.int32),
            pltpu.VMEM((CH, H), jnp.float32),
            pltpu.VMEM((CH, H), jnp.float32),
            pltpu.VMEM((TAIL, H), jnp.float32),
            pltpu.SemaphoreType.DMA,
            pltpu.SemaphoreType.DMA,
        ],
        compiler_params=pltpu.CompilerParams(use_tc_tiling_on_sc=False),
    )
    def k(t_hbm, ei_hbm, pre_hbm, idxd, idxs, buf0, buf1, buf_t,
          sem0, sem1):
        c = lax.axis_index("c")
        s = lax.axis_index("s")
        wid = s * NC + c
        base0 = wid * EPW

        pltpu.sync_copy(ei_hbm.at[pl.ds(E + base0, EPW)], idxd)
        pltpu.sync_copy(ei_hbm.at[pl.ds(base0, EPW)], idxs)

        def shift(i, carry):
            sl = pl.ds(i * 16, 16)
            idxd[sl] = idxd[sl] * 2
            idxs[sl] = idxs[sl] * 2 + 1
            return carry

        lax.fori_loop(0, EPW // 16, shift, 0)

        def gsta(ci, b, sem, add):
            iref = idxs if add else idxd
            pltpu.async_copy(t_hbm.at[iref.at[pl.ds(ci * CH, CH)]], b, sem,
                             add=add)

        def gwait(ci, b, sem, add):
            iref = idxs if add else idxd
            pltpu.make_async_copy(t_hbm.at[iref.at[pl.ds(ci * CH, CH)]], b,
                                  sem).wait()

        def ssta(ci, b, sem):
            pltpu.async_copy(b, pre_hbm.at[pl.ds(base0 + ci * CH, CH)], sem)

        def swait(ci, b, sem):
            pltpu.make_async_copy(b, pre_hbm.at[pl.ds(base0 + ci * CH, CH)],
                                  sem).wait()

        NP = NFULL // 2
        gsta(0, buf0, sem0, False)

        def body(i, carry):
            c0 = 2 * i
            c1 = c0 + 1
            gwait(c0, buf0, sem0, False)
            gsta(c0, buf0, sem0, True)

            @pl.when(i > 0)
            def _():
                swait(c1 - 2, buf1, sem1)

            gsta(c1, buf1, sem1, False)
            gwait(c0, buf0, sem0, True)
            ssta(c0, buf0, sem0)
            gwait(c1, buf1, sem1, False)
            gsta(c1, buf1, sem1, True)
            swait(c0, buf0, sem0)

            @pl.when(i < NP - 1)
            def _():
                gsta(c0 + 2, buf0, sem0, False)

            gwait(c1, buf1, sem1, True)
            ssta(c1, buf1, sem1)
            return carry

        lax.fori_loop(0, NP, body, 0)
        swait(NFULL - 1, buf1, sem1)

        tb = NFULL * CH
        pltpu.sync_copy(t_hbm.at[idxd.at[pl.ds(tb, TAIL)]], buf_t)
        pltpu.sync_copy(t_hbm.at[idxs.at[pl.ds(tb, TAIL)]], buf_t, add=True)
        pltpu.sync_copy(buf_t, pre_hbm.at[pl.ds(base0 + tb, TAIL)])

    return k(T, ei1d)


DW = 16


def _sc_scatter(h2, ei1d, with_deg):
    out_type = [jax.ShapeDtypeStruct((NC, N, H), jnp.float32)]
    scratch = [
        pltpu.VMEM_SHARED((N, H), jnp.float32),
        pltpu.VMEM((ROWS_PER_SUB, H), jnp.float32),
        pltpu.VMEM((CH,), jnp.int32),
        pltpu.VMEM((CH,), jnp.int32),
        pltpu.VMEM((CH, H), jnp.float32),
        pltpu.VMEM((CH, H), jnp.float32),
        pltpu.VMEM((TAIL,), jnp.int32),
        pltpu.VMEM((TAIL, H), jnp.float32),
        pltpu.SemaphoreType.DMA,
        pltpu.SemaphoreType.DMA,
        pltpu.SemaphoreType.DMA,
        pltpu.SemaphoreType.DMA,
    ]
    if with_deg:
        out_type.append(jax.ShapeDtypeStruct((NC, N, DW), jnp.float32))
        scratch += [
            pltpu.VMEM_SHARED((N, DW), jnp.float32),
            pltpu.VMEM((ROWS_PER_SUB, DW), jnp.float32),
            pltpu.VMEM((CH, DW), jnp.float32),
            pltpu.VMEM((TAIL, DW), jnp.float32),
        ]

    @functools.partial(
        pl.kernel,
        out_type=tuple(out_type),
        mesh=plsc.VectorSubcoreMesh(**_MESH),
        scratch_types=scratch,
        compiler_params=pltpu.CompilerParams(use_tc_tiling_on_sc=False),
    )
    def k(h2_hbm, ei_hbm, out_hbm, *rest):
        if with_deg:
            (outd_hbm, acc, stage, idx0, idx1, buf0, buf1, idx_t, buf_t,
             semL0, semL1, semW0, semW1, dacc, dstage, ones, ones_t) = rest
        else:
            (acc, stage, idx0, idx1, buf0, buf1, idx_t, buf_t,
             semL0, semL1, semW0, semW1) = rest
        c = lax.axis_index("c")
        s = lax.axis_index("s")
        wid = s * NC + c
        base0 = wid * EPW
        eibase = E
        rsl = pl.ds(s * ROWS_PER_SUB, ROWS_PER_SUB)

        def zrow(i, carry):
            for j in range(H // 16):
                stage[i, pl.ds(j * 16, 16)] = jnp.zeros((16,), jnp.float32)
            return carry

        lax.fori_loop(0, ROWS_PER_SUB, zrow, 0)
        pltpu.sync_copy(stage, acc.at[rsl])
        if with_deg:
            def zdrow(i, carry):
                dstage[i, pl.ds(0, 16)] = jnp.zeros((16,), jnp.float32)
                return carry

            lax.fori_loop(0, ROWS_PER_SUB, zdrow, 0)
            pltpu.sync_copy(dstage, dacc.at[rsl])

            def orow(i, carry):
                ones[i, pl.ds(0, 16)] = jnp.ones((16,), jnp.float32)
                return carry

            lax.fori_loop(0, CH, orow, 0)
            for i in range(TAIL):
                ones_t[i, pl.ds(0, 16)] = jnp.ones((16,), jnp.float32)
        plsc.subcore_barrier()

        def lsta(ci, i_r, b, sem):
            base = base0 + ci * CH
            pltpu.async_copy(ei_hbm.at[pl.ds(eibase + base, CH)], i_r, sem)
            pltpu.async_copy(h2_hbm.at[pl.ds(base, CH)], b, sem)

        def lwait(ci, i_r, b, sem):
            base = base0 + ci * CH
            pltpu.make_async_copy(ei_hbm.at[pl.ds(eibase + base, CH)], i_r,
                                  sem).wait()
            pltpu.make_async_copy(h2_hbm.at[pl.ds(base, CH)], b, sem).wait()

        def wsta(i_r, b, sem):
            pltpu.async_copy(b, acc.at[i_r], sem, add=True)

        def wwait(i_r, b, sem):
            pltpu.make_async_copy(b, acc.at[i_r], sem).wait()

        NP = NFULL // 2
        lsta(0, idx0, buf0, semL0)

        def body(i, carry):
            c0 = 2 * i
            c1 = c0 + 1
            lwait(c0, idx0, buf0, semL0)

            @pl.when(i > 0)
            def _():
                wwait(idx1, buf1, semW1)

            lsta(c1, idx1, buf1, semL1)
            wsta(idx0, buf0, semW0)
            if with_deg:
                pltpu.sync_copy(ones, dacc.at[idx0], add=True)
            lwait(c1, idx1, buf1, semL1)
            wwait(idx0, buf0, semW0)

            @pl.when(i < NP - 1)
            def _():
                lsta(c0 + 2, idx0, buf0, semL0)

            wsta(idx1, buf1, semW1)
            if with_deg:
                pltpu.sync_copy(ones, dacc.at[idx1], add=True)
            return carry

        lax.fori_loop(0, NP, body, 0)
        wwait(idx1, buf1, semW1)

        tb = base0 + NFULL * CH
        pltpu.sync_copy(ei_hbm.at[pl.ds(eibase + tb, TAIL)], idx_t)
        pltpu.sync_copy(h2_hbm.at[pl.ds(tb, TAIL)], buf_t)
        pltpu.sync_copy(buf_t, acc.at[idx_t], add=True)
        if with_deg:
            pltpu.sync_copy(ones_t, dacc.at[idx_t], add=True)

        plsc.subcore_barrier()
        pltpu.sync_copy(acc.at[rsl], stage)
        pltpu.sync_copy(stage, out_hbm.at[c, rsl])
        if with_deg:
            pltpu.sync_copy(dacc.at[rsl], dstage)
            pltpu.sync_copy(dstage, outd_hbm.at[c, rsl])

    res = k(h2, ei1d)
    return res if with_deg else res[0]


def _prep1(x, w1):
    BR, NB = 2000, N // 2000

    def body(x_ref, w_ref, o_ref):
        w = w_ref[...]
        wb = w[D:]
        wcat = jnp.concatenate([w[:D] - wb, wb], axis=1)
        o_ref[...] = jnp.dot(x_ref[...], wcat,
                             preferred_element_type=jnp.float32)

    return pl.pallas_call(
        body,
        grid=(NB,),
        in_specs=[
            pl.BlockSpec((BR, D), lambda j: (j, 0)),
            pl.BlockSpec((2 * D, H), lambda j: (0, 0)),
        ],
        out_specs=pl.BlockSpec((BR, 2 * H), lambda j: (j, 0)),
        out_shape=jax.ShapeDtypeStruct((N, 2 * H), jnp.float32),
    )(x, w1)


def _prep2(S, w1, s2d, crow, degp):
    BR, NB = 2000, N // 2000

    def body(s_ref, w_ref, sc_ref, c_ref, d_ref, o_ref):
        xin = s_ref[0] + s_ref[1]
        dcol = d_ref[0, :, 0:1] + d_ref[1, :, 0:1]
        w = w_ref[...]
        wb = w[H:]
        wcat = jnp.concatenate([w[:H] - wb, wb], axis=1)
        vrow = jnp.dot(c_ref[...], wcat, preferred_element_type=jnp.float32)
        o_ref[...] = (jnp.dot(xin, sc_ref[...] * wcat,
                              preferred_element_type=jnp.float32)
                      + dcol * vrow)

    return pl.pallas_call(
        body,
        grid=(NB,),
        in_specs=[
            pl.BlockSpec((2, BR, H), lambda j: (0, j, 0)),
            pl.BlockSpec((2 * H, H), lambda j: (0, 0)),
            pl.BlockSpec((H, 1), lambda j: (0, 0)),
            pl.BlockSpec((1, H), lambda j: (0, 0)),
            pl.BlockSpec((2, BR, DW), lambda j: (0, j, 0)),
        ],
        out_specs=pl.BlockSpec((BR, 2 * H), lambda j: (j, 0)),
        out_shape=jax.ShapeDtypeStruct((N, 2 * H), jnp.float32),
    )(S, w1, s2d, crow, degp)


def _edge_stats(pre128, b1row):
    BR = 8000
    E2 = E // 2
    NB = E2 // BR

    def body(p_ref, b_ref, o_ref):
        b128 = jnp.concatenate([b_ref[...], b_ref[...]], axis=1)
        h = jnp.maximum(p_ref[...] + b128, 0.0)

        @pl.when(pl.program_id(0) == 0)
        def _():
            o_ref[...] = jnp.zeros_like(o_ref)

        o_ref[0:1, :] += jnp.sum(h, axis=0, keepdims=True)
        o_ref[1:2, :] += jnp.sum(h * h, axis=0, keepdims=True)

    return pl.pallas_call(
        body,
        grid=(NB,),
        in_specs=[
            pl.BlockSpec((BR, 2 * H), lambda i: (i, 0)),
            pl.BlockSpec((1, H), lambda i: (0, 0)),
        ],
        out_specs=pl.BlockSpec((8, 2 * H), lambda i: (0, 0)),
        out_shape=jax.ShapeDtypeStruct((8, 2 * H), jnp.float32),
    )(pre128, b1row)


def _edge_mm(pre128, b1row, w2, s1d, c1row, b2row):
    BR = 8000
    E2 = E // 2
    NB = E2 // BR

    def body(p_ref, b1_ref, w_ref, s1_ref, c1_ref, b2_ref, h2_ref, st_ref):
        w2p = s1_ref[...] * w_ref[...]
        b2p = (jnp.dot(c1_ref[...], w_ref[...],
                       preferred_element_type=jnp.float32) + b2_ref[...])
        b128 = jnp.concatenate([b1_ref[...], b1_ref[...]], axis=1)
        h1 = jnp.maximum(p_ref[...] + b128, 0.0)
        zz = jnp.zeros((H, H), jnp.float32)
        w2bd = jnp.concatenate(
            [jnp.concatenate([w2p, zz], axis=1),
             jnp.concatenate([zz, w2p], axis=1)], axis=0)
        h2 = jnp.maximum(
            jnp.dot(h1, w2bd, preferred_element_type=jnp.float32)
            + jnp.concatenate([b2p, b2p], axis=1), 0.0)
        h2_ref[...] = h2

        @pl.when(pl.program_id(0) == 0)
        def _():
            st_ref[...] = jnp.zeros_like(st_ref)

        st_ref[0:1, :] += jnp.sum(h2, axis=0, keepdims=True)
        st_ref[1:2, :] += jnp.sum(h2 * h2, axis=0, keepdims=True)

    return pl.pallas_call(
        body,
        grid=(NB,),
        in_specs=[
            pl.BlockSpec((BR, 2 * H), lambda i: (i, 0)),
            pl.BlockSpec((1, H), lambda i: (0, 0)),
            pl.BlockSpec((H, H), lambda i: (0, 0)),
            pl.BlockSpec((H, 1), lambda i: (0, 0)),
            pl.BlockSpec((1, H), lambda i: (0, 0)),
            pl.BlockSpec((1, H), lambda i: (0, 0)),
        ],
        out_specs=[
            pl.BlockSpec((BR, 2 * H), lambda i: (i, 0)),
            pl.BlockSpec((8, 2 * H), lambda i: (0, 0)),
        ],
        out_shape=[
            jax.ShapeDtypeStruct((E2, 2 * H), jnp.float32),
            jax.ShapeDtypeStruct((8, 2 * H), jnp.float32),
        ],
    )(pre128, b1row, w2, s1d, c1row, b2row)


def _cat_mm(S1, S2, degp, w, brow, sa, sb, ca, cb):
    BR, NB = 2000, N // 2000
    MH = w.shape[1]

    def body(s1_ref, s2_ref, d_ref, w_ref, b_ref, sa_ref, sb_ref, ca_ref,
             cb_ref, r_ref, st_ref):
        x1 = s1_ref[0] + s1_ref[1]
        x2 = s2_ref[0] + s2_ref[1]
        dcol = d_ref[0, :, 0:1] + d_ref[1, :, 0:1]
        wt = w_ref[...][:H]
        wb = w_ref[...][H:]
        vrow = (jnp.dot(ca_ref[...], wt, preferred_element_type=jnp.float32)
                + jnp.dot(cb_ref[...], wb,
                          preferred_element_type=jnp.float32))
        z = (jnp.dot(x1, sa_ref[...] * wt,
                     preferred_element_type=jnp.float32)
             + jnp.dot(x2, sb_ref[...] * wb,
                       preferred_element_type=jnp.float32)
             + dcol * vrow + b_ref[...])
        r = jnp.maximum(z, 0.0)
        r_ref[...] = r

        @pl.when(pl.program_id(0) == 0)
        def _():
            st_ref[...] = jnp.zeros_like(st_ref)

        st_ref[0:1, :] += jnp.sum(r, axis=0, keepdims=True)
        st_ref[1:2, :] += jnp.sum(r * r, axis=0, keepdims=True)

    return pl.pallas_call(
        body,
        grid=(NB,),
        in_specs=[
            pl.BlockSpec((2, BR, H), lambda j: (0, j, 0)),
            pl.BlockSpec((2, BR, H), lambda j: (0, j, 0)),
            pl.BlockSpec((2, BR, DW), lambda j: (0, j, 0)),
            pl.BlockSpec((2 * H, MH), lambda j: (0, 0)),
            pl.BlockSpec((1, MH), lambda j: (0, 0)),
            pl.BlockSpec((H, 1), lambda j: (0, 0)),
            pl.BlockSpec((H, 1), lambda j: (0, 0)),
            pl.BlockSpec((1, H), lambda j: (0, 0)),
            pl.BlockSpec((1, H), lambda j: (0, 0)),
        ],
        out_specs=[
            pl.BlockSpec((BR, MH), lambda j: (j, 0)),
            pl.BlockSpec((8, MH), lambda j: (0, 0)),
        ],
        out_shape=[
            jax.ShapeDtypeStruct((N, MH), jnp.float32),
            jax.ShapeDtypeStruct((8, MH), jnp.float32),
        ],
    )(S1, S2, degp, w, brow, sa, sb, ca, cb)


def _node_mm(xin, w, brow, sd, crow):
    BR, NB = 2000, N // 2000
    K, M = w.shape

    def body(x_ref, w_ref, b_ref, s_ref, c_ref, r_ref, st_ref):
        w = w_ref[...]
        beff = (jnp.dot(c_ref[...], w, preferred_element_type=jnp.float32)
                + b_ref[...])
        r = jnp.maximum(
            jnp.dot(x_ref[...], s_ref[...] * w,
                    preferred_element_type=jnp.float32) + beff, 0.0)
        r_ref[...] = r

        @pl.when(pl.program_id(0) == 0)
        def _():
            st_ref[...] = jnp.zeros_like(st_ref)

        st_ref[0:1, :] += jnp.sum(r, axis=0, keepdims=True)
        st_ref[1:2, :] += jnp.sum(r * r, axis=0, keepdims=True)

    return pl.pallas_call(
        body,
        grid=(NB,),
        in_specs=[
            pl.BlockSpec((BR, K), lambda j: (j, 0)),
            pl.BlockSpec((K, M), lambda j: (0, 0)),
            pl.BlockSpec((1, M), lambda j: (0, 0)),
            pl.BlockSpec((K, 1), lambda j: (0, 0)),
            pl.BlockSpec((1, K), lambda j: (0, 0)),
        ],
        out_specs=[
            pl.BlockSpec((BR, M), lambda j: (j, 0)),
            pl.BlockSpec((8, M), lambda j: (0, 0)),
        ],
        out_shape=[
            jax.ShapeDtypeStruct((N, M), jnp.float32),
            jax.ShapeDtypeStruct((8, M), jnp.float32),
        ],
    )(xin, w, brow, sd, crow)


def _final_mm(xin, w, brow, sd, crow):
    BR, NB = 2000, N // 2000
    K, M = w.shape

    def body(x_ref, w_ref, b_ref, s_ref, c_ref, o_ref):
        w = w_ref[...]
        beff = (jnp.dot(c_ref[...], w, preferred_element_type=jnp.float32)
                + b_ref[...])
        z = jnp.dot(x_ref[...], s_ref[...] * w,
                    preferred_element_type=jnp.float32) + beff
        m = jnp.max(z, axis=1, keepdims=True)
        lse = jnp.log(jnp.sum(jnp.exp(z - m), axis=1, keepdims=True)) + m
        o_ref[...] = z - lse

    return pl.pallas_call(
        body,
        grid=(NB,),
        in_specs=[
            pl.BlockSpec((BR, K), lambda j: (j, 0)),
            pl.BlockSpec((K, M), lambda j: (0, 0)),
            pl.BlockSpec((1, M), lambda j: (0, 0)),
            pl.BlockSpec((K, 1), lambda j: (0, 0)),
            pl.BlockSpec((1, K), lambda j: (0, 0)),
        ],
        out_specs=pl.BlockSpec((BR, M), lambda j: (j, 0)),
        out_shape=jax.ShapeDtypeStruct((N, M), jnp.float32),
    )(xin, w, brow, sd, crow)


def _bn_fold(st, g, be):
    mu = st[0] / E
    var = st[1] / E - mu * mu
    s = g / jnp.sqrt(var + EPS)
    return mu, s, be - s * mu


def _bn_fold_n(st, g, be):
    mu = st[0] / N
    var = st[1] / N - mu * mu
    s = g / jnp.sqrt(var + EPS)
    return mu, s, be - s * mu


def _edge_layer(xin_T, ei1d, b1, g1, be1, W2, b2, g2, be2, with_deg):
    pre = _sc_gather(xin_T, ei1d)
    pre128 = pre.reshape(E // 2, 2 * H)
    b1r = b1.reshape(1, H)
    st1p = _edge_stats(pre128, b1r)
    st1 = st1p[:, :H] + st1p[:, H:]
    _, s1, c1 = _bn_fold(st1, g1, be1)
    s1d = s1.reshape(H, 1)
    c1r = c1.reshape(1, H)
    b2r = b2.reshape(1, H)
    h2_128, st2p = _edge_mm(pre128, b1r, W2, s1d, c1r, b2r)
    st2 = st2p[:, :H] + st2p[:, H:]
    mu2 = st2[0] / E
    var2 = st2[1] / E - mu2 * mu2
    s2 = g2 / jnp.sqrt(var2 + EPS)
    c2 = be2 - s2 * mu2
    out = _sc_scatter(h2_128.reshape(E, H), ei1d, with_deg)
    if with_deg:
        S, degp = out
        return S, s2, c2, degp
    return out, s2, c2


def kernel(x, edge_index, c1_W1, c1_b1, c1_g1, c1_be1, c1_W2, c1_b2, c1_g2,
           c1_be2, c2_W1, c2_b1, c2_g1, c2_be1, c2_W2, c2_b2, c2_g2, c2_be2,
           l1_W, l1_b, l1_g, l1_be, m1_W, m1_b, m1_g, m1_be, m2_W, m2_b,
           m2_g, m2_be, f_W, f_b):
    ei1d = edge_index.reshape(2 * E)

    T1 = _prep1(x, c1_W1).reshape(2 * N, H)
    S1, s2a, c2a, degp = _edge_layer(T1, ei1d, c1_b1, c1_g1, c1_be1,
                                     c1_W2, c1_b2, c1_g2, c1_be2, True)

    T2 = _prep2(S1, c2_W1, s2a.reshape(H, 1), c2a.reshape(1, H),
                degp).reshape(2 * N, H)
    S2, s2b, c2b = _edge_layer(T2, ei1d, c2_b1, c2_g1, c2_be1,
                               c2_W2, c2_b2, c2_g2, c2_be2, False)

    r1, stA = _cat_mm(S1, S2, degp, l1_W, l1_b.reshape(1, -1),
                      s2a.reshape(H, 1), s2b.reshape(H, 1),
                      c2a.reshape(1, H), c2b.reshape(1, H))
    _, sA, cA = _bn_fold_n(stA, l1_g, l1_be)

    r2, stB = _node_mm(r1, m1_W, m1_b.reshape(1, -1),
                       sA.reshape(-1, 1), cA.reshape(1, -1))
    _, sB, cB = _bn_fold_n(stB, m1_g, m1_be)

    r3, stC = _node_mm(r2, m2_W, m2_b.reshape(1, -1),
                       sB.reshape(-1, 1), cB.reshape(1, -1))
    _, sC, cC = _bn_fold_n(stC, m2_g, m2_be)

    return _final_mm(r3, f_W, f_b.reshape(1, -1),
                     sC.reshape(-1, 1), cC.reshape(1, -1))


# speedup vs baseline: 1.0749x; 1.0096x over previous
import functools

import jax
import jax.numpy as jnp
from jax import lax
from jax.experimental import pallas as pl
from jax.experimental.pallas import tpu as pltpu
from jax.experimental.pallas import tpu_sc as plsc

N = 10000
E = 320000
D = 128
H = 64
EPS = 1e-5

NC = 2
NS = 16
NW = NC * NS
CH = 128
EPW = E // NW
NFULL = EPW // CH
TAIL = EPW - NFULL * CH

_MESH = dict(core_axis_name="c", subcore_axis_name="s", num_cores=NC,
             num_subcores=NS)

ROWS_PER_SUB = N // NS


def _sc_gather(T, ei1d):
    @functools.partial(
        pl.kernel,
        out_type=jax.ShapeDtypeStruct((E, H), jnp.float32),
        mesh=plsc.VectorSubcoreMesh(**_MESH),
        scratch_types=[
            pltpu.VMEM((EPW,), jnp.int32),
            pltpu.VMEM((EPW,), jnp.int32),
            pltpu.VMEM((CH, H), jnp.float32),
            pltpu.VMEM((CH, H), jnp.float32),
            pltpu.VMEM((TAIL, H), jnp.float32),
            pltpu.SemaphoreType.DMA,
            pltpu.SemaphoreType.DMA,
        ],
        compiler_params=pltpu.CompilerParams(use_tc_tiling_on_sc=False),
    )
    def k(t_hbm, ei_hbm, pre_hbm, idxd, idxs, buf0, buf1, buf_t,
          sem0, sem1):
        c = lax.axis_index("c")
        s = lax.axis_index("s")
        wid = s * NC + c
        base0 = wid * EPW

        pltpu.sync_copy(ei_hbm.at[pl.ds(E + base0, EPW)], idxd)
        pltpu.sync_copy(ei_hbm.at[pl.ds(base0, EPW)], idxs)

        def shift(i, carry):
            sl = pl.ds(i * 16, 16)
            idxd[sl] = idxd[sl] * 2
            idxs[sl] = idxs[sl] * 2 + 1
            return carry

        lax.fori_loop(0, EPW // 16, shift, 0)

        def gsta(ci, b, sem, add):
            iref = idxs if add else idxd
            pltpu.async_copy(t_hbm.at[iref.at[pl.ds(ci * CH, CH)]], b, sem,
                             add=add)

        def gwait(ci, b, sem, add):
            iref = idxs if add else idxd
            pltpu.make_async_copy(t_hbm.at[iref.at[pl.ds(ci * CH, CH)]], b,
                                  sem).wait()

        def ssta(ci, b, sem):
            pltpu.async_copy(b, pre_hbm.at[pl.ds(base0 + ci * CH, CH)], sem)

        def swait(ci, b, sem):
            pltpu.make_async_copy(b, pre_hbm.at[pl.ds(base0 + ci * CH, CH)],
                                  sem).wait()

        NP = NFULL // 2
        gsta(0, buf0, sem0, False)

        def body(i, carry):
            c0 = 2 * i
            c1 = c0 + 1
            gwait(c0, buf0, sem0, False)
            gsta(c0, buf0, sem0, True)

            @pl.when(i > 0)
            def _():
                swait(c1 - 2, buf1, sem1)

            gsta(c1, buf1, sem1, False)
            gwait(c0, buf0, sem0, True)
            ssta(c0, buf0, sem0)
            gwait(c1, buf1, sem1, False)
            gsta(c1, buf1, sem1, True)
            swait(c0, buf0, sem0)

            @pl.when(i < NP - 1)
            def _():
                gsta(c0 + 2, buf0, sem0, False)

            gwait(c1, buf1, sem1, True)
            ssta(c1, buf1, sem1)
            return carry

        lax.fori_loop(0, NP, body, 0)
        swait(NFULL - 1, buf1, sem1)

        tb = NFULL * CH
        pltpu.sync_copy(t_hbm.at[idxd.at[pl.ds(tb, TAIL)]], buf_t)
        pltpu.sync_copy(t_hbm.at[idxs.at[pl.ds(tb, TAIL)]], buf_t, add=True)
        pltpu.sync_copy(buf_t, pre_hbm.at[pl.ds(base0 + tb, TAIL)])

    return k(T, ei1d)


DW = 16


def _sc_scatter(h2, ei1d, with_deg):
    out_type = [jax.ShapeDtypeStruct((NC, N, H), jnp.float32)]
    scratch = [
        pltpu.VMEM_SHARED((N, H), jnp.float32),
        pltpu.VMEM((ROWS_PER_SUB, H), jnp.float32),
        pltpu.VMEM((CH,), jnp.int32),
        pltpu.VMEM((CH,), jnp.int32),
        pltpu.VMEM((CH, H), jnp.float32),
        pltpu.VMEM((CH, H), jnp.float32),
        pltpu.VMEM((TAIL,), jnp.int32),
        pltpu.VMEM((TAIL, H), jnp.float32),
        pltpu.SemaphoreType.DMA,
        pltpu.SemaphoreType.DMA,
        pltpu.SemaphoreType.DMA,
        pltpu.SemaphoreType.DMA,
    ]
    if with_deg:
        out_type.append(jax.ShapeDtypeStruct((NC, N, DW), jnp.float32))
        scratch += [
            pltpu.VMEM_SHARED((N, DW), jnp.float32),
            pltpu.VMEM((ROWS_PER_SUB, DW), jnp.float32),
            pltpu.VMEM((CH, DW), jnp.float32),
            pltpu.VMEM((TAIL, DW), jnp.float32),
        ]

    @functools.partial(
        pl.kernel,
        out_type=tuple(out_type),
        mesh=plsc.VectorSubcoreMesh(**_MESH),
        scratch_types=scratch,
        compiler_params=pltpu.CompilerParams(use_tc_tiling_on_sc=False),
    )
    def k(h2_hbm, ei_hbm, out_hbm, *rest):
        if with_deg:
            (outd_hbm, acc, stage, idx0, idx1, buf0, buf1, idx_t, buf_t,
             semL0, semL1, semW0, semW1, dacc, dstage, ones, ones_t) = rest
        else:
            (acc, stage, idx0, idx1, buf0, buf1, idx_t, buf_t,
             semL0, semL1, semW0, semW1) = rest
        c = lax.axis_index("c")
        s = lax.axis_index("s")
        wid = s * NC + c
        base0 = wid * EPW
        eibase = E
        rsl = pl.ds(s * ROWS_PER_SUB, ROWS_PER_SUB)

        def zrow(i, carry):
            for j in range(H // 16):
                stage[i, pl.ds(j * 16, 16)] = jnp.zeros((16,), jnp.float32)
            return carry

        lax.fori_loop(0, ROWS_PER_SUB, zrow, 0)
        pltpu.sync_copy(stage, acc.at[rsl])
        if with_deg:
            def zdrow(i, carry):
                dstage[i, pl.ds(0, 16)] = jnp.zeros((16,), jnp.float32)
                return carry

            lax.fori_loop(0, ROWS_PER_SUB, zdrow, 0)
            pltpu.sync_copy(dstage, dacc.at[rsl])

            def orow(i, carry):
                ones[i, pl.ds(0, 16)] = jnp.ones((16,), jnp.float32)
                return carry

            lax.fori_loop(0, CH, orow, 0)
            for i in range(TAIL):
                ones_t[i, pl.ds(0, 16)] = jnp.ones((16,), jnp.float32)
        plsc.subcore_barrier()

        def lsta(ci, i_r, b, sem):
            base = base0 + ci * CH
            pltpu.async_copy(ei_hbm.at[pl.ds(eibase + base, CH)], i_r, sem)
            pltpu.async_copy(h2_hbm.at[pl.ds(base, CH)], b, sem)

        def lwait(ci, i_r, b, sem):
            base = base0 + ci * CH
            pltpu.make_async_copy(ei_hbm.at[pl.ds(eibase + base, CH)], i_r,
                                  sem).wait()
            pltpu.make_async_copy(h2_hbm.at[pl.ds(base, CH)], b, sem).wait()

        def wsta(i_r, b, sem):
            pltpu.async_copy(b, acc.at[i_r], sem, add=True)

        def wwait(i_r, b, sem):
            pltpu.make_async_copy(b, acc.at[i_r], sem).wait()

        NP = NFULL // 2
        lsta(0, idx0, buf0, semL0)

        def body(i, carry):
            c0 = 2 * i
            c1 = c0 + 1
            lwait(c0, idx0, buf0, semL0)

            @pl.when(i > 0)
            def _():
                wwait(idx1, buf1, semW1)

            lsta(c1, idx1, buf1, semL1)
            wsta(idx0, buf0, semW0)
            if with_deg:
                pltpu.sync_copy(ones, dacc.at[idx0], add=True)
            lwait(c1, idx1, buf1, semL1)
            wwait(idx0, buf0, semW0)

            @pl.when(i < NP - 1)
            def _():
                lsta(c0 + 2, idx0, buf0, semL0)

            wsta(idx1, buf1, semW1)
            if with_deg:
                pltpu.sync_copy(ones, dacc.at[idx1], add=True)
            return carry

        lax.fori_loop(0, NP, body, 0)
        wwait(idx1, buf1, semW1)

        tb = base0 + NFULL * CH
        pltpu.sync_copy(ei_hbm.at[pl.ds(eibase + tb, TAIL)], idx_t)
        pltpu.sync_copy(h2_hbm.at[pl.ds(tb, TAIL)], buf_t)
        pltpu.sync_copy(buf_t, acc.at[idx_t], add=True)
        if with_deg:
            pltpu.sync_copy(ones_t, dacc.at[idx_t], add=True)

        plsc.subcore_barrier()
        pltpu.sync_copy(acc.at[rsl], stage)
        pltpu.sync_copy(stage, out_hbm.at[c, rsl])
        if with_deg:
            pltpu.sync_copy(dacc.at[rsl], dstage)
            pltpu.sync_copy(dstage, outd_hbm.at[c, rsl])

    res = k(h2, ei1d)
    return res if with_deg else res[0]


def _prep1(x, w1):
    BR, NB = 2000, N // 2000

    def body(x_ref, w_ref, o_ref):
        w = w_ref[...]
        wb = w[D:]
        wcat = jnp.concatenate([w[:D] - wb, wb], axis=1)
        o_ref[...] = jnp.dot(x_ref[...], wcat,
                             preferred_element_type=jnp.float32)

    return pl.pallas_call(
        body,
        grid=(NB,),
        in_specs=[
            pl.BlockSpec((BR, D), lambda j: (j, 0)),
            pl.BlockSpec((2 * D, H), lambda j: (0, 0)),
        ],
        out_specs=pl.BlockSpec((BR, 2 * H), lambda j: (j, 0)),
        out_shape=jax.ShapeDtypeStruct((N, 2 * H), jnp.float32),
    )(x, w1)


def _prep2(S, w1, s2d, crow, degp):
    BR, NB = 2000, N // 2000

    def body(s_ref, w_ref, sc_ref, c_ref, d_ref, o_ref):
        xin = s_ref[0] + s_ref[1]
        dcol = d_ref[0, :, 0:1] + d_ref[1, :, 0:1]
        w = w_ref[...]
        wb = w[H:]
        wcat = jnp.concatenate([w[:H] - wb, wb], axis=1)
        vrow = jnp.dot(c_ref[...], wcat, preferred_element_type=jnp.float32)
        o_ref[...] = (jnp.dot(xin, sc_ref[...] * wcat,
                              preferred_element_type=jnp.float32)
                      + dcol * vrow)

    return pl.pallas_call(
        body,
        grid=(NB,),
        in_specs=[
            pl.BlockSpec((2, BR, H), lambda j: (0, j, 0)),
            pl.BlockSpec((2 * H, H), lambda j: (0, 0)),
            pl.BlockSpec((H, 1), lambda j: (0, 0)),
            pl.BlockSpec((1, H), lambda j: (0, 0)),
            pl.BlockSpec((2, BR, DW), lambda j: (0, j, 0)),
        ],
        out_specs=pl.BlockSpec((BR, 2 * H), lambda j: (j, 0)),
        out_shape=jax.ShapeDtypeStruct((N, 2 * H), jnp.float32),
    )(S, w1, s2d, crow, degp)


def _edge_stats(pre128, b1row):
    BR = 16000
    E2 = E // 2
    NB = E2 // BR

    def body(p_ref, b_ref, o_ref):
        b128 = jnp.concatenate([b_ref[...], b_ref[...]], axis=1)
        h = jnp.maximum(p_ref[...] + b128, 0.0)

        @pl.when(pl.program_id(0) == 0)
        def _():
            o_ref[...] = jnp.zeros_like(o_ref)

        o_ref[0:1, :] += jnp.sum(h, axis=0, keepdims=True)
        o_ref[1:2, :] += jnp.sum(h * h, axis=0, keepdims=True)

    return pl.pallas_call(
        body,
        grid=(NB,),
        in_specs=[
            pl.BlockSpec((BR, 2 * H), lambda i: (i, 0)),
            pl.BlockSpec((1, H), lambda i: (0, 0)),
        ],
        out_specs=pl.BlockSpec((8, 2 * H), lambda i: (0, 0)),
        out_shape=jax.ShapeDtypeStruct((8, 2 * H), jnp.float32),
    )(pre128, b1row)


def _edge_mm(pre128, b1row, w2, s1d, c1row, b2row):
    BR = 8000
    E2 = E // 2
    NB = E2 // BR

    def body(p_ref, b1_ref, w_ref, s1_ref, c1_ref, b2_ref, h2_ref, st_ref):
        w2p = s1_ref[...] * w_ref[...]
        b2p = (jnp.dot(c1_ref[...], w_ref[...],
                       preferred_element_type=jnp.float32) + b2_ref[...])
        b128 = jnp.concatenate([b1_ref[...], b1_ref[...]], axis=1)
        h1 = jnp.maximum(p_ref[...] + b128, 0.0)
        zz = jnp.zeros((H, H), jnp.float32)
        w2bd = jnp.concatenate(
            [jnp.concatenate([w2p, zz], axis=1),
             jnp.concatenate([zz, w2p], axis=1)], axis=0)
        h2 = jnp.maximum(
            jnp.dot(h1, w2bd, preferred_element_type=jnp.float32)
            + jnp.concatenate([b2p, b2p], axis=1), 0.0)
        h2_ref[...] = h2

        @pl.when(pl.program_id(0) == 0)
        def _():
            st_ref[...] = jnp.zeros_like(st_ref)

        st_ref[0:1, :] += jnp.sum(h2, axis=0, keepdims=True)
        st_ref[1:2, :] += jnp.sum(h2 * h2, axis=0, keepdims=True)

    return pl.pallas_call(
        body,
        grid=(NB,),
        in_specs=[
            pl.BlockSpec((BR, 2 * H), lambda i: (i, 0)),
            pl.BlockSpec((1, H), lambda i: (0, 0)),
            pl.BlockSpec((H, H), lambda i: (0, 0)),
            pl.BlockSpec((H, 1), lambda i: (0, 0)),
            pl.BlockSpec((1, H), lambda i: (0, 0)),
            pl.BlockSpec((1, H), lambda i: (0, 0)),
        ],
        out_specs=[
            pl.BlockSpec((BR, 2 * H), lambda i: (i, 0)),
            pl.BlockSpec((8, 2 * H), lambda i: (0, 0)),
        ],
        out_shape=[
            jax.ShapeDtypeStruct((E2, 2 * H), jnp.float32),
            jax.ShapeDtypeStruct((8, 2 * H), jnp.float32),
        ],
    )(pre128, b1row, w2, s1d, c1row, b2row)


def _cat_mm(S1, S2, degp, w, brow, sa, sb, ca, cb):
    BR, NB = 2000, N // 2000
    MH = w.shape[1]

    def body(s1_ref, s2_ref, d_ref, w_ref, b_ref, sa_ref, sb_ref, ca_ref,
             cb_ref, r_ref, st_ref):
        x1 = s1_ref[0] + s1_ref[1]
        x2 = s2_ref[0] + s2_ref[1]
        dcol = d_ref[0, :, 0:1] + d_ref[1, :, 0:1]
        wt = w_ref[...][:H]
        wb = w_ref[...][H:]
        vrow = (jnp.dot(ca_ref[...], wt, preferred_element_type=jnp.float32)
                + jnp.dot(cb_ref[...], wb,
                          preferred_element_type=jnp.float32))
        z = (jnp.dot(x1, sa_ref[...] * wt,
                     preferred_element_type=jnp.float32)
             + jnp.dot(x2, sb_ref[...] * wb,
                       preferred_element_type=jnp.float32)
             + dcol * vrow + b_ref[...])
        r = jnp.maximum(z, 0.0)
        r_ref[...] = r

        @pl.when(pl.program_id(0) == 0)
        def _():
            st_ref[...] = jnp.zeros_like(st_ref)

        st_ref[0:1, :] += jnp.sum(r, axis=0, keepdims=True)
        st_ref[1:2, :] += jnp.sum(r * r, axis=0, keepdims=True)

    return pl.pallas_call(
        body,
        grid=(NB,),
        in_specs=[
            pl.BlockSpec((2, BR, H), lambda j: (0, j, 0)),
            pl.BlockSpec((2, BR, H), lambda j: (0, j, 0)),
            pl.BlockSpec((2, BR, DW), lambda j: (0, j, 0)),
            pl.BlockSpec((2 * H, MH), lambda j: (0, 0)),
            pl.BlockSpec((1, MH), lambda j: (0, 0)),
            pl.BlockSpec((H, 1), lambda j: (0, 0)),
            pl.BlockSpec((H, 1), lambda j: (0, 0)),
            pl.BlockSpec((1, H), lambda j: (0, 0)),
            pl.BlockSpec((1, H), lambda j: (0, 0)),
        ],
        out_specs=[
            pl.BlockSpec((BR, MH), lambda j: (j, 0)),
            pl.BlockSpec((8, MH), lambda j: (0, 0)),
        ],
        out_shape=[
            jax.ShapeDtypeStruct((N, MH), jnp.float32),
            jax.ShapeDtypeStruct((8, MH), jnp.float32),
        ],
    )(S1, S2, degp, w, brow, sa, sb, ca, cb)


def _node_mm(xin, w, brow, sd, crow):
    BR, NB = 2000, N // 2000
    K, M = w.shape

    def body(x_ref, w_ref, b_ref, s_ref, c_ref, r_ref, st_ref):
        w = w_ref[...]
        beff = (jnp.dot(c_ref[...], w, preferred_element_type=jnp.float32)
                + b_ref[...])
        r = jnp.maximum(
            jnp.dot(x_ref[...], s_ref[...] * w,
                    preferred_element_type=jnp.float32) + beff, 0.0)
        r_ref[...] = r

        @pl.when(pl.program_id(0) == 0)
        def _():
            st_ref[...] = jnp.zeros_like(st_ref)

        st_ref[0:1, :] += jnp.sum(r, axis=0, keepdims=True)
        st_ref[1:2, :] += jnp.sum(r * r, axis=0, keepdims=True)

    return pl.pallas_call(
        body,
        grid=(NB,),
        in_specs=[
            pl.BlockSpec((BR, K), lambda j: (j, 0)),
            pl.BlockSpec((K, M), lambda j: (0, 0)),
            pl.BlockSpec((1, M), lambda j: (0, 0)),
            pl.BlockSpec((K, 1), lambda j: (0, 0)),
            pl.BlockSpec((1, K), lambda j: (0, 0)),
        ],
        out_specs=[
            pl.BlockSpec((BR, M), lambda j: (j, 0)),
            pl.BlockSpec((8, M), lambda j: (0, 0)),
        ],
        out_shape=[
            jax.ShapeDtypeStruct((N, M), jnp.float32),
            jax.ShapeDtypeStruct((8, M), jnp.float32),
        ],
    )(xin, w, brow, sd, crow)


def _final_mm(xin, w, brow, sd, crow):
    BR, NB = 2000, N // 2000
    K, M = w.shape

    def body(x_ref, w_ref, b_ref, s_ref, c_ref, o_ref):
        w = w_ref[...]
        beff = (jnp.dot(c_ref[...], w, preferred_element_type=jnp.float32)
                + b_ref[...])
        z = jnp.dot(x_ref[...], s_ref[...] * w,
                    preferred_element_type=jnp.float32) + beff
        m = jnp.max(z, axis=1, keepdims=True)
        lse = jnp.log(jnp.sum(jnp.exp(z - m), axis=1, keepdims=True)) + m
        o_ref[...] = z - lse

    return pl.pallas_call(
        body,
        grid=(NB,),
        in_specs=[
            pl.BlockSpec((BR, K), lambda j: (j, 0)),
            pl.BlockSpec((K, M), lambda j: (0, 0)),
            pl.BlockSpec((1, M), lambda j: (0, 0)),
            pl.BlockSpec((K, 1), lambda j: (0, 0)),
            pl.BlockSpec((1, K), lambda j: (0, 0)),
        ],
        out_specs=pl.BlockSpec((BR, M), lambda j: (j, 0)),
        out_shape=jax.ShapeDtypeStruct((N, M), jnp.float32),
    )(xin, w, brow, sd, crow)


def _bn_fold(st, g, be):
    mu = st[0] / E
    var = st[1] / E - mu * mu
    s = g / jnp.sqrt(var + EPS)
    return mu, s, be - s * mu


def _bn_fold_n(st, g, be):
    mu = st[0] / N
    var = st[1] / N - mu * mu
    s = g / jnp.sqrt(var + EPS)
    return mu, s, be - s * mu


def _edge_layer(xin_T, ei1d, b1, g1, be1, W2, b2, g2, be2, with_deg):
    pre = _sc_gather(xin_T, ei1d)
    pre128 = pre.reshape(E // 2, 2 * H)
    b1r = b1.reshape(1, H)
    st1p = _edge_stats(pre128, b1r)
    st1 = st1p[:, :H] + st1p[:, H:]
    _, s1, c1 = _bn_fold(st1, g1, be1)
    s1d = s1.reshape(H, 1)
    c1r = c1.reshape(1, H)
    b2r = b2.reshape(1, H)
    h2_128, st2p = _edge_mm(pre128, b1r, W2, s1d, c1r, b2r)
    st2 = st2p[:, :H] + st2p[:, H:]
    mu2 = st2[0] / E
    var2 = st2[1] / E - mu2 * mu2
    s2 = g2 / jnp.sqrt(var2 + EPS)
    c2 = be2 - s2 * mu2
    out = _sc_scatter(h2_128.reshape(E, H), ei1d, with_deg)
    if with_deg:
        S, degp = out
        return S, s2, c2, degp
    return out, s2, c2


def kernel(x, edge_index, c1_W1, c1_b1, c1_g1, c1_be1, c1_W2, c1_b2, c1_g2,
           c1_be2, c2_W1, c2_b1, c2_g1, c2_be1, c2_W2, c2_b2, c2_g2, c2_be2,
           l1_W, l1_b, l1_g, l1_be, m1_W, m1_b, m1_g, m1_be, m2_W, m2_b,
           m2_g, m2_be, f_W, f_b):
    ei1d = edge_index.reshape(2 * E)

    T1 = _prep1(x, c1_W1).reshape(2 * N, H)
    S1, s2a, c2a, degp = _edge_layer(T1, ei1d, c1_b1, c1_g1, c1_be1,
                                     c1_W2, c1_b2, c1_g2, c1_be2, True)

    T2 = _prep2(S1, c2_W1, s2a.reshape(H, 1), c2a.reshape(1, H),
                degp).reshape(2 * N, H)
    S2, s2b, c2b = _edge_layer(T2, ei1d, c2_b1, c2_g1, c2_be1,
                               c2_W2, c2_b2, c2_g2, c2_be2, False)

    r1, stA = _cat_mm(S1, S2, degp, l1_W, l1_b.reshape(1, -1),
                      s2a.reshape(H, 1), s2b.reshape(H, 1),
                      c2a.reshape(1, H), c2b.reshape(1, H))
    _, sA, cA = _bn_fold_n(stA, l1_g, l1_be)

    r2, stB = _node_mm(r1, m1_W, m1_b.reshape(1, -1),
                       sA.reshape(-1, 1), cA.reshape(1, -1))
    _, sB, cB = _bn_fold_n(stB, m1_g, m1_be)

    r3, stC = _node_mm(r2, m2_W, m2_b.reshape(1, -1),
                       sB.reshape(-1, 1), cB.reshape(1, -1))
    _, sC, cC = _bn_fold_n(stC, m2_g, m2_be)

    return _final_mm(r3, f_W, f_b.reshape(1, -1),
                     sC.reshape(-1, 1), cC.reshape(1, -1))
